# baseline (speedup 1.0000x reference)
import jax
import jax.numpy as jnp
from jax import lax
from jax.experimental import pallas as pl
from jax.experimental.pallas import tpu as pltpu
from jax.experimental.pallas import tpu_sc as plsc

D_MODEL = 64
N_BATCH = 4096
SEQ = 200
NUM_CORES = 2
NUM_SUBCORES = 16
NUM_WORKERS = NUM_CORES * NUM_SUBCORES
BLK = N_BATCH // NUM_WORKERS
LANES = 16
NBUF = 2
TRW = 129


def _embed_body(xt_hbm, table_hbm, out_hbm, idx_t, rows0, rows1, tr0, tr1,
                gsem, ssem):
    rows_b = (rows0, rows1)
    tr_b = (tr0, tr1)
    wid = lax.axis_index("s") * NUM_CORES + lax.axis_index("c")
    b0 = wid * BLK

    iota = lax.broadcasted_iota(jnp.int32, (LANES,), 0)
    dvecs = [iota + g * LANES for g in range(D_MODEL // LANES)]

    pltpu.sync_copy(xt_hbm.at[:, pl.ds(b0, BLK)], idx_t)

    def gather_cp(s, b):
        return pltpu.make_async_copy(
            table_hbm.at[idx_t.at[s]], rows_b[b], gsem.at[b])

    def store_cp(s, b, r8):
        return pltpu.make_async_copy(
            tr_b[b].at[pl.ds(r8 * 8, 8), pl.ds(0, 128)],
            out_hbm.at[s, r8, wid], ssem.at[b])

    UNROLL = 4

    def transpose_rows(b):
        def lstep(lq, carry):
            for u in range(UNROLL):
                l = lq * UNROLL + u
                sp = jnp.full((LANES,), l, dtype=jnp.int32)
                for g in range(D_MODEL // LANES):
                    v = rows_b[b][l, pl.ds(g * LANES, LANES)]
                    plsc.store_scatter(tr_b[b], [dvecs[g], sp], v)
            return carry

        lax.fori_loop(0, BLK // UNROLL, lstep, 0)

    for b in range(NBUF):
        gather_cp(b, b).start()

    def step(g, carry):
        for b in range(NBUF):
            s = g * NBUF + b
            gather_cp(s, b).wait()

            @pl.when(s >= NBUF)
            def _():
                for r8 in range(D_MODEL // 8):
                    store_cp(s, b, r8).wait()

            transpose_rows(b)

            @pl.when(s + NBUF < SEQ)
            def _():
                gather_cp(s + NBUF, b).start()

            for r8 in range(D_MODEL // 8):
                store_cp(s, b, r8).start()

        return carry

    lax.fori_loop(0, SEQ // NBUF, step, 0)

    for b in range(NBUF):
        for r8 in range(D_MODEL // 8):
            store_cp(SEQ - NBUF + b, b, r8).wait()


@jax.jit
def kernel(x, weight):
    xt = (x.astype(jnp.int32) * 2).T
    table = jnp.concatenate([weight, weight], axis=1).reshape(2000000, 64)
    mesh = plsc.VectorSubcoreMesh(core_axis_name="c", subcore_axis_name="s")
    out5 = pl.kernel(
        _embed_body,
        out_type=jax.ShapeDtypeStruct(
            (SEQ, D_MODEL // 8, NUM_WORKERS, 8, 128), jnp.float32),
        mesh=mesh,
        scratch_types=[
            pltpu.VMEM((SEQ, BLK), jnp.int32),
            pltpu.VMEM((BLK, D_MODEL), jnp.float32),
            pltpu.VMEM((BLK, D_MODEL), jnp.float32),
            pltpu.VMEM((D_MODEL, TRW), jnp.float32),
            pltpu.VMEM((D_MODEL, TRW), jnp.float32),
            pltpu.SemaphoreType.DMA((NBUF,)),
            pltpu.SemaphoreType.DMA((NBUF,)),
        ],
        compiler_params=pltpu.CompilerParams(
            use_tc_tiling_on_sc=False, needs_layout_passes=False),
    )(xt, table)
    return out5.transpose(2, 4, 0, 1, 3).reshape(N_BATCH, SEQ, D_MODEL)


# speedup vs baseline: 1.3385x; 1.3385x over previous
import jax
import jax.numpy as jnp
from jax import lax
from jax.experimental import pallas as pl
from jax.experimental.pallas import tpu as pltpu
from jax.experimental.pallas import tpu_sc as plsc

D_MODEL = 64
N_BATCH = 4096
SEQ = 200
NUM_CORES = 2
NUM_SUBCORES = 16
NUM_WORKERS = NUM_CORES * NUM_SUBCORES
BLK = N_BATCH // NUM_WORKERS
LANES = 16
NBUF = 2
TRW = 129


def _embed_body(xt_hbm, table_hbm, out_hbm, idx_t, rows0, rows1, tr0, tr1,
                gsem, ssem):
    rows_b = (rows0, rows1)
    tr_b = (tr0, tr1)
    wid = lax.axis_index("s") * NUM_CORES + lax.axis_index("c")
    b0 = wid * BLK

    iota = lax.broadcasted_iota(jnp.int32, (LANES,), 0)
    dvecs = [iota + g * LANES for g in range(D_MODEL // LANES)]

    pltpu.sync_copy(xt_hbm.at[:, pl.ds(b0, BLK)], idx_t)

    def gather_cp(s, b):
        return pltpu.make_async_copy(
            table_hbm.at[idx_t.at[s]], rows_b[b], gsem.at[b])

    def store_cp(s, b, r8):
        return pltpu.make_async_copy(
            tr_b[b].at[pl.ds(r8 * 8, 8), pl.ds(0, 128)],
            out_hbm.at[s, r8, wid], ssem.at[b])

    UNROLL = 4

    def transpose_rows(b):
        def lstep(lq, carry):
            vals = []
            for u in range(UNROLL):
                l = lq * UNROLL + u
                sp = jnp.full((LANES,), l, dtype=jnp.int32)
                for g in range(D_MODEL // LANES):
                    vals.append(
                        (rows_b[b][l, pl.ds(g * LANES, LANES)], dvecs[g], sp))
            for v, dv, sp in vals:
                plsc.store_scatter(tr_b[b], [dv, sp], v)
            return carry

        lax.fori_loop(0, BLK // UNROLL, lstep, 0)

    for b in range(NBUF):
        gather_cp(b, b).start()

    def step(g, carry):
        for b in range(NBUF):
            s = g * NBUF + b
            gather_cp(s, b).wait()

            @pl.when(s >= NBUF)
            def _():
                for r8 in range(D_MODEL // 8):
                    store_cp(s, b, r8).wait()

            transpose_rows(b)

            @pl.when(s + NBUF < SEQ)
            def _():
                gather_cp(s + NBUF, b).start()

            for r8 in range(D_MODEL // 8):
                store_cp(s, b, r8).start()

        return carry

    lax.fori_loop(0, SEQ // NBUF, step, 0)

    for b in range(NBUF):
        for r8 in range(D_MODEL // 8):
            store_cp(SEQ - NBUF + b, b, r8).wait()


@jax.jit
def kernel(x, weight):
    xt = (x.astype(jnp.int32) * 2).T
    table = jnp.pad(weight, ((0, 0), (0, 64))).reshape(2000000, 64)
    mesh = plsc.VectorSubcoreMesh(core_axis_name="c", subcore_axis_name="s")
    out5 = pl.kernel(
        _embed_body,
        out_type=jax.ShapeDtypeStruct(
            (SEQ, D_MODEL // 8, NUM_WORKERS, 8, 128), jnp.float32),
        mesh=mesh,
        scratch_types=[
            pltpu.VMEM((SEQ, BLK), jnp.int32),
            pltpu.VMEM((BLK, D_MODEL), jnp.float32),
            pltpu.VMEM((BLK, D_MODEL), jnp.float32),
            pltpu.VMEM((D_MODEL, TRW), jnp.float32),
            pltpu.VMEM((D_MODEL, TRW), jnp.float32),
            pltpu.SemaphoreType.DMA((NBUF,)),
            pltpu.SemaphoreType.DMA((NBUF,)),
        ],
        compiler_params=pltpu.CompilerParams(
            use_tc_tiling_on_sc=False, needs_layout_passes=False),
    )(xt, table)
    return out5.transpose(2, 4, 0, 1, 3).reshape(N_BATCH, SEQ, D_MODEL)
